# baseline (speedup 1.0000x reference)
import functools

import jax
import jax.numpy as jnp
from jax import lax
from jax.experimental import pallas as pl
from jax.experimental.pallas import tpu as pltpu
from jax.experimental.pallas import tpu_sc as plsc

KNN = 16
LANES = 16
NUM_WORKERS = 32
COLS_AT_ONCE = 4


def _merge_sorted(run_v, run_i, new_v, new_i):
  sv, si = plsc.sort_key_val(new_v, new_i)
  sv = lax.rev(sv, (0,))
  si = lax.rev(si, (0,))
  keep = run_v <= sv
  lo_v = jnp.where(keep, run_v, sv)
  lo_i = jnp.where(keep, run_i, si)
  out_v, out_i = plsc.sort_key_val(lo_v, lo_i)
  return out_v, out_i


@functools.lru_cache(maxsize=None)
def _make_knn(b_sz, n_pts, k_cen):
  assert NUM_WORKERS % b_sz == 0
  workers_per_batch = NUM_WORKERS // b_sz
  cols_per_worker = k_cen // workers_per_batch
  assert cols_per_worker * workers_per_batch == k_cen
  assert cols_per_worker % COLS_AT_ONCE == 0
  blk_pts = 16 * LANES
  num_blocks = n_pts // blk_pts
  assert num_blocks * blk_pts == n_pts

  mesh = plsc.VectorSubcoreMesh(core_axis_name="c", subcore_axis_name="s")

  @functools.partial(
      pl.kernel,
      out_type=jax.ShapeDtypeStruct((b_sz * KNN, k_cen), jnp.int32),
      mesh=mesh,
      scratch_types=[
          pltpu.VMEM((n_pts,), jnp.float32),
          pltpu.VMEM((n_pts,), jnp.float32),
          pltpu.VMEM((n_pts,), jnp.float32),
          pltpu.VMEM((n_pts,), jnp.float32),
          pltpu.VMEM((cols_per_worker,), jnp.float32),
          pltpu.VMEM((cols_per_worker,), jnp.float32),
          pltpu.VMEM((cols_per_worker,), jnp.float32),
          pltpu.VMEM((KNN, cols_per_worker), jnp.int32),
      ],
      compiler_params=pltpu.CompilerParams(needs_layout_passes=False),
  )
  def knn(x_h, y_h, z_h, cx_h, cy_h, cz_h, out_h,
          x_v, y_v, z_v, s_v, cx_v, cy_v, cz_v, out_v):
    wid = lax.axis_index("s") * 2 + lax.axis_index("c")
    b = wid // workers_per_batch
    c0 = (wid % workers_per_batch) * cols_per_worker

    pltpu.sync_copy(x_h.at[pl.ds(b * n_pts, n_pts)], x_v)
    pltpu.sync_copy(y_h.at[pl.ds(b * n_pts, n_pts)], y_v)
    pltpu.sync_copy(z_h.at[pl.ds(b * n_pts, n_pts)], z_v)
    pltpu.sync_copy(cx_h.at[pl.ds(b * k_cen + c0, cols_per_worker)], cx_v)
    pltpu.sync_copy(cy_h.at[pl.ds(b * k_cen + c0, cols_per_worker)], cy_v)
    pltpu.sync_copy(cz_h.at[pl.ds(b * k_cen + c0, cols_per_worker)], cz_v)

    @pl.loop(0, n_pts // LANES)
    def norm_loop(i):
      sl = pl.ds(i * LANES, LANES)
      xv = x_v[sl]
      yv = y_v[sl]
      zv = z_v[sl]
      s_v[sl] = xv * xv + yv * yv + zv * zv

    lane_iota = lax.iota(jnp.int32, LANES)
    inf_v = jnp.full((LANES,), jnp.inf, jnp.float32)
    zero_i = jnp.zeros((LANES,), jnp.int32)
    nc = COLS_AT_ONCE

    @pl.loop(0, cols_per_worker, step=nc)
    def col_loop(cl):
      tx, ty, tz = [], [], []
      for j in range(nc):
        cj = jnp.full((LANES,), cl + j, jnp.int32)
        cxs = plsc.load_gather(cx_v, [cj])
        cys = plsc.load_gather(cy_v, [cj])
        czs = plsc.load_gather(cz_v, [cj])
        tx.append(cxs + cxs)
        ty.append(cys + cys)
        tz.append(czs + czs)

      def blk_body(t, carry):
        base = t * blk_pts
        m = [None] * nc
        for i in range(16):
          sl = pl.ds(base + i * LANES, LANES)
          xv = x_v[sl]
          yv = y_v[sl]
          zv = z_v[sl]
          sv = s_v[sl]
          for j in range(nc):
            d2 = sv - (xv * tx[j] + yv * ty[j] + zv * tz[j])
            m[j] = d2 if m[j] is None else jnp.minimum(m[j], d2)
        ids = lane_iota + t * LANES
        return tuple(
            _merge_sorted(carry[j][0], carry[j][1], m[j], ids)
            for j in range(nc))

      cands = lax.fori_loop(0, num_blocks, blk_body,
                            tuple((inf_v, zero_i) for _ in range(nc)))

      for j in range(nc):
        cand = cands[j][1]
        pbase = ((cand >> 4) * blk_pts) + (cand & (LANES - 1))
        fin_v, fin_i = inf_v, zero_i
        for i in range(16):
          pidx = pbase + i * LANES
          xg = plsc.load_gather(x_v, [pidx])
          yg = plsc.load_gather(y_v, [pidx])
          zg = plsc.load_gather(z_v, [pidx])
          sg = plsc.load_gather(s_v, [pidx])
          d2 = sg - (xg * tx[j] + yg * ty[j] + zg * tz[j])
          fin_v, fin_i = _merge_sorted(fin_v, fin_i, d2, pidx)

        plsc.store_scatter(
            out_v, [lane_iota, jnp.full((LANES,), cl + j, jnp.int32)], fin_i)

    pltpu.sync_copy(
        out_v, out_h.at[pl.ds(b * KNN, KNN), pl.ds(c0, cols_per_worker)])

  return knn


def kernel(xyz, centers):
  b_sz, n_pts, _ = xyz.shape
  k_cen = centers.shape[1]
  knn = _make_knn(b_sz, n_pts, k_cen)
  pts = jnp.transpose(xyz, (2, 0, 1)).reshape(3, b_sz * n_pts)
  cen = jnp.transpose(centers, (2, 0, 1)).reshape(3, b_sz * k_cen)
  out2d = knn(pts[0], pts[1], pts[2], cen[0], cen[1], cen[2])
  return out2d.reshape(b_sz, KNN, k_cen)


# speedup vs baseline: 2.0115x; 2.0115x over previous
import functools

import jax
import jax.numpy as jnp
from jax import lax
from jax.experimental import pallas as pl
from jax.experimental.pallas import tpu as pltpu
from jax.experimental.pallas import tpu_sc as plsc

KNN = 16
LANES = 16
NUM_WORKERS = 32
COLS_AT_ONCE = 2


def _merge_sorted(run_v, run_i, new_v, new_i):
  sv, si = plsc.sort_key_val(new_v, new_i)
  sv = lax.rev(sv, (0,))
  si = lax.rev(si, (0,))
  keep = run_v <= sv
  lo_v = jnp.where(keep, run_v, sv)
  lo_i = jnp.where(keep, run_i, si)
  out_v, out_i = plsc.sort_key_val(lo_v, lo_i)
  return out_v, out_i


@functools.lru_cache(maxsize=None)
def _make_knn(b_sz, n_pts, k_cen):
  assert NUM_WORKERS % b_sz == 0
  workers_per_batch = NUM_WORKERS // b_sz
  cols_per_worker = k_cen // workers_per_batch
  assert cols_per_worker * workers_per_batch == k_cen
  assert cols_per_worker % COLS_AT_ONCE == 0
  blk_pts = 16 * LANES
  num_blocks = n_pts // blk_pts
  assert num_blocks * blk_pts == n_pts

  mesh = plsc.VectorSubcoreMesh(core_axis_name="c", subcore_axis_name="s")

  @functools.partial(
      pl.kernel,
      out_type=jax.ShapeDtypeStruct((b_sz * KNN, k_cen), jnp.int32),
      mesh=mesh,
      scratch_types=[
          pltpu.VMEM((n_pts,), jnp.float32),
          pltpu.VMEM((n_pts,), jnp.float32),
          pltpu.VMEM((n_pts,), jnp.float32),
          pltpu.VMEM((n_pts,), jnp.float32),
          pltpu.VMEM((cols_per_worker,), jnp.float32),
          pltpu.VMEM((cols_per_worker,), jnp.float32),
          pltpu.VMEM((cols_per_worker,), jnp.float32),
          pltpu.VMEM((KNN, cols_per_worker), jnp.int32),
      ],
      compiler_params=pltpu.CompilerParams(needs_layout_passes=False),
  )
  def knn(x_h, y_h, z_h, cx_h, cy_h, cz_h, out_h,
          x_v, y_v, z_v, s_v, cx_v, cy_v, cz_v, out_v):
    wid = lax.axis_index("s") * 2 + lax.axis_index("c")
    b = wid // workers_per_batch
    c0 = (wid % workers_per_batch) * cols_per_worker

    pltpu.sync_copy(x_h.at[pl.ds(b * n_pts, n_pts)], x_v)
    pltpu.sync_copy(y_h.at[pl.ds(b * n_pts, n_pts)], y_v)
    pltpu.sync_copy(z_h.at[pl.ds(b * n_pts, n_pts)], z_v)
    pltpu.sync_copy(cx_h.at[pl.ds(b * k_cen + c0, cols_per_worker)], cx_v)
    pltpu.sync_copy(cy_h.at[pl.ds(b * k_cen + c0, cols_per_worker)], cy_v)
    pltpu.sync_copy(cz_h.at[pl.ds(b * k_cen + c0, cols_per_worker)], cz_v)

    @pl.loop(0, n_pts // LANES)
    def norm_loop(i):
      sl = pl.ds(i * LANES, LANES)
      xv = x_v[sl]
      yv = y_v[sl]
      zv = z_v[sl]
      s_v[sl] = xv * xv + yv * yv + zv * zv

    lane_iota = lax.iota(jnp.int32, LANES)
    inf_v = jnp.full((LANES,), jnp.inf, jnp.float32)
    zero_i = jnp.zeros((LANES,), jnp.int32)
    nc = COLS_AT_ONCE

    @pl.loop(0, cols_per_worker, step=nc)
    def col_loop(cl):
      tx, ty, tz = [], [], []
      for j in range(nc):
        cj = jnp.full((LANES,), cl + j, jnp.int32)
        cxs = plsc.load_gather(cx_v, [cj])
        cys = plsc.load_gather(cy_v, [cj])
        czs = plsc.load_gather(cz_v, [cj])
        tx.append(cxs + cxs)
        ty.append(cys + cys)
        tz.append(czs + czs)

      def blk_body(t, carry):
        base = t * blk_pts
        m = [None] * nc
        for i in range(16):
          sl = pl.ds(base + i * LANES, LANES)
          xv = x_v[sl]
          yv = y_v[sl]
          zv = z_v[sl]
          sv = s_v[sl]
          for j in range(nc):
            d2 = sv - (xv * tx[j] + yv * ty[j] + zv * tz[j])
            m[j] = d2 if m[j] is None else jnp.minimum(m[j], d2)
        ids = lane_iota + t * LANES
        return tuple(
            _merge_sorted(carry[j][0], carry[j][1], m[j], ids)
            for j in range(nc))

      cands = lax.fori_loop(0, num_blocks, blk_body,
                            tuple((inf_v, zero_i) for _ in range(nc)))

      for j in range(nc):
        cand = cands[j][1]
        pbase = ((cand >> 4) * blk_pts) + (cand & (LANES - 1))
        cxs = jnp.float32(0.5) * tx[j]
        cys = jnp.float32(0.5) * ty[j]
        czs = jnp.float32(0.5) * tz[j]
        fin_v, fin_i = inf_v, zero_i
        for i in range(16):
          pidx = pbase + i * LANES
          dx = plsc.load_gather(x_v, [pidx]) - cxs
          dy = plsc.load_gather(y_v, [pidx]) - cys
          dz = plsc.load_gather(z_v, [pidx]) - czs
          d2 = dx * dx + dy * dy + dz * dz
          fin_v, fin_i = _merge_sorted(fin_v, fin_i, d2, pidx)

        plsc.store_scatter(
            out_v, [lane_iota, jnp.full((LANES,), cl + j, jnp.int32)], fin_i)

    pltpu.sync_copy(
        out_v, out_h.at[pl.ds(b * KNN, KNN), pl.ds(c0, cols_per_worker)])

  return knn


def kernel(xyz, centers):
  b_sz, n_pts, _ = xyz.shape
  k_cen = centers.shape[1]
  knn = _make_knn(b_sz, n_pts, k_cen)
  pts = jnp.transpose(xyz, (2, 0, 1)).reshape(3, b_sz * n_pts)
  cen = jnp.transpose(centers, (2, 0, 1)).reshape(3, b_sz * k_cen)
  out2d = knn(pts[0], pts[1], pts[2], cen[0], cen[1], cen[2])
  return out2d.reshape(b_sz, KNN, k_cen)


# speedup vs baseline: 2.8145x; 1.3992x over previous
import functools

import jax
import jax.numpy as jnp
from jax import lax
from jax.experimental import pallas as pl
from jax.experimental.pallas import tpu as pltpu
from jax.experimental.pallas import tpu_sc as plsc

KNN = 16
LANES = 16
NUM_WORKERS = 32


def _merge_sorted(run_v, run_i, new_v, new_i):
  sv, si = plsc.sort_key_val(new_v, new_i)
  return _merge_two_sorted((run_v, run_i), (sv, si))


def _merge_two_sorted(a, b):
  av, ai = a
  bv, bi = b
  bv = lax.rev(bv, (0,))
  bi = lax.rev(bi, (0,))
  keep = av <= bv
  lo_v = jnp.where(keep, av, bv)
  lo_i = jnp.where(keep, ai, bi)
  out_v, out_i = plsc.sort_key_val(lo_v, lo_i)
  return out_v, out_i


@functools.lru_cache(maxsize=None)
def _make_knn(b_sz, n_pts, k_cen):
  assert NUM_WORKERS % b_sz == 0
  workers_per_batch = NUM_WORKERS // b_sz
  cols_per_worker = k_cen // workers_per_batch
  assert cols_per_worker * workers_per_batch == k_cen
  blk_pts = 16 * LANES
  num_blocks = n_pts // blk_pts
  assert num_blocks * blk_pts == n_pts

  mesh = plsc.VectorSubcoreMesh(core_axis_name="c", subcore_axis_name="s")

  @functools.partial(
      pl.kernel,
      out_type=jax.ShapeDtypeStruct((b_sz * KNN, k_cen), jnp.int32),
      mesh=mesh,
      scratch_types=[
          pltpu.VMEM((n_pts,), jnp.float32),
          pltpu.VMEM((n_pts,), jnp.float32),
          pltpu.VMEM((n_pts,), jnp.float32),
          pltpu.VMEM((cols_per_worker,), jnp.float32),
          pltpu.VMEM((cols_per_worker,), jnp.float32),
          pltpu.VMEM((cols_per_worker,), jnp.float32),
          pltpu.VMEM((KNN, cols_per_worker), jnp.int32),
      ],
      compiler_params=pltpu.CompilerParams(needs_layout_passes=False),
  )
  def knn(x_h, y_h, z_h, cx_h, cy_h, cz_h, out_h,
          x_v, y_v, z_v, cx_v, cy_v, cz_v, out_v):
    wid = lax.axis_index("s") * 2 + lax.axis_index("c")
    b = wid // workers_per_batch
    c0 = (wid % workers_per_batch) * cols_per_worker

    pltpu.sync_copy(x_h.at[pl.ds(b * n_pts, n_pts)], x_v)
    pltpu.sync_copy(y_h.at[pl.ds(b * n_pts, n_pts)], y_v)
    pltpu.sync_copy(z_h.at[pl.ds(b * n_pts, n_pts)], z_v)
    pltpu.sync_copy(cx_h.at[pl.ds(b * k_cen + c0, cols_per_worker)], cx_v)
    pltpu.sync_copy(cy_h.at[pl.ds(b * k_cen + c0, cols_per_worker)], cy_v)
    pltpu.sync_copy(cz_h.at[pl.ds(b * k_cen + c0, cols_per_worker)], cz_v)

    lane_iota = lax.iota(jnp.int32, LANES)
    inf_v = jnp.full((LANES,), jnp.inf, jnp.float32)
    zero_i = jnp.zeros((LANES,), jnp.int32)

    @pl.loop(0, cols_per_worker)
    def col_loop(cl):
      cl_v = jnp.full((LANES,), cl, jnp.int32)
      cx = plsc.load_gather(cx_v, [cl_v])
      cy = plsc.load_gather(cy_v, [cl_v])
      cz = plsc.load_gather(cz_v, [cl_v])

      def blk_body(t, carry):
        run_v, run_i = carry
        base = t * blk_pts
        m = None
        for i in range(16):
          sl = pl.ds(base + i * LANES, LANES)
          dx = x_v[sl] - cx
          dy = y_v[sl] - cy
          dz = z_v[sl] - cz
          d2 = dx * dx + dy * dy + dz * dz
          m = d2 if m is None else jnp.minimum(m, d2)

        def do_merge(rv, ri, mm):
          return _merge_sorted(rv, ri, mm, lane_iota + t * LANES)

        return lax.cond(jnp.min(m) < run_v[15], do_merge,
                        lambda rv, ri, mm: (rv, ri), run_v, run_i, m)

      _, cand = lax.fori_loop(0, num_blocks, blk_body, (inf_v, zero_i))

      pbase = ((cand >> 4) * blk_pts) + (cand & (LANES - 1))
      groups = []
      for i in range(16):
        pidx = pbase + i * LANES
        dx = plsc.load_gather(x_v, [pidx]) - cx
        dy = plsc.load_gather(y_v, [pidx]) - cy
        dz = plsc.load_gather(z_v, [pidx]) - cz
        d2 = dx * dx + dy * dy + dz * dz
        sv, si = plsc.sort_key_val(d2, pidx)
        groups.append((sv, si))
      while len(groups) > 1:
        groups = [_merge_two_sorted(groups[k], groups[k + 1])
                  for k in range(0, len(groups), 2)]
      fin_i = groups[0][1]

      plsc.store_scatter(out_v, [lane_iota, cl_v], fin_i)

    pltpu.sync_copy(
        out_v, out_h.at[pl.ds(b * KNN, KNN), pl.ds(c0, cols_per_worker)])

  return knn


def kernel(xyz, centers):
  b_sz, n_pts, _ = xyz.shape
  k_cen = centers.shape[1]
  knn = _make_knn(b_sz, n_pts, k_cen)
  pts = jnp.transpose(xyz, (2, 0, 1)).reshape(3, b_sz * n_pts)
  cen = jnp.transpose(centers, (2, 0, 1)).reshape(3, b_sz * k_cen)
  out2d = knn(pts[0], pts[1], pts[2], cen[0], cen[1], cen[2])
  return out2d.reshape(b_sz, KNN, k_cen)


# speedup vs baseline: 3.1063x; 1.1037x over previous
import functools

import jax
import jax.numpy as jnp
from jax import lax
from jax.experimental import pallas as pl
from jax.experimental.pallas import tpu as pltpu
from jax.experimental.pallas import tpu_sc as plsc

KNN = 16
LANES = 16
NUM_WORKERS = 32


def _merge_sorted(run_v, run_i, new_v, new_i):
  sv, si = plsc.sort_key_val(new_v, new_i)
  return _merge_two_sorted((run_v, run_i), (sv, si))


def _merge_two_sorted(a, b):
  av, ai = a
  bv, bi = b
  bv = lax.rev(bv, (0,))
  bi = lax.rev(bi, (0,))
  keep = av <= bv
  lo_v = jnp.where(keep, av, bv)
  lo_i = jnp.where(keep, ai, bi)
  out_v, out_i = plsc.sort_key_val(lo_v, lo_i)
  return out_v, out_i


@functools.lru_cache(maxsize=None)
def _make_knn(b_sz, n_pts, k_cen):
  assert NUM_WORKERS % b_sz == 0
  workers_per_batch = NUM_WORKERS // b_sz
  cols_per_worker = k_cen // workers_per_batch
  assert cols_per_worker * workers_per_batch == k_cen
  blk_pts = 16 * LANES
  num_blocks = n_pts // blk_pts
  assert num_blocks * blk_pts == n_pts

  mesh = plsc.VectorSubcoreMesh(core_axis_name="c", subcore_axis_name="s")

  @functools.partial(
      pl.kernel,
      out_type=jax.ShapeDtypeStruct((b_sz * KNN, k_cen), jnp.int32),
      mesh=mesh,
      scratch_types=[
          pltpu.VMEM((n_pts,), jnp.float32),
          pltpu.VMEM((n_pts,), jnp.float32),
          pltpu.VMEM((n_pts,), jnp.float32),
          pltpu.VMEM((cols_per_worker,), jnp.float32),
          pltpu.VMEM((cols_per_worker,), jnp.float32),
          pltpu.VMEM((cols_per_worker,), jnp.float32),
          pltpu.VMEM((KNN, cols_per_worker), jnp.int32),
      ],
      compiler_params=pltpu.CompilerParams(needs_layout_passes=False),
  )
  def knn(x_h, y_h, z_h, cx_h, cy_h, cz_h, out_h,
          x_v, y_v, z_v, cx_v, cy_v, cz_v, out_v):
    wid = lax.axis_index("s") * 2 + lax.axis_index("c")
    b = wid // workers_per_batch
    c0 = (wid % workers_per_batch) * cols_per_worker

    pltpu.sync_copy(x_h.at[pl.ds(b * n_pts, n_pts)], x_v)
    pltpu.sync_copy(y_h.at[pl.ds(b * n_pts, n_pts)], y_v)
    pltpu.sync_copy(z_h.at[pl.ds(b * n_pts, n_pts)], z_v)
    pltpu.sync_copy(cx_h.at[pl.ds(b * k_cen + c0, cols_per_worker)], cx_v)
    pltpu.sync_copy(cy_h.at[pl.ds(b * k_cen + c0, cols_per_worker)], cy_v)
    pltpu.sync_copy(cz_h.at[pl.ds(b * k_cen + c0, cols_per_worker)], cz_v)

    lane_iota = lax.iota(jnp.int32, LANES)
    inf_v = jnp.full((LANES,), jnp.inf, jnp.float32)
    zero_i = jnp.zeros((LANES,), jnp.int32)

    @pl.loop(0, cols_per_worker)
    def col_loop(cl):
      cl_v = jnp.full((LANES,), cl, jnp.int32)
      cx = plsc.load_gather(cx_v, [cl_v])
      cy = plsc.load_gather(cy_v, [cl_v])
      cz = plsc.load_gather(cz_v, [cl_v])

      def blk_body(t, carry):
        run_v, run_i = carry
        base = t * blk_pts
        m = None
        for i in range(16):
          sl = pl.ds(base + i * LANES, LANES)
          dx = x_v[sl] - cx
          dy = y_v[sl] - cy
          dz = z_v[sl] - cz
          d2 = dx * dx + dy * dy + dz * dz
          m = d2 if m is None else jnp.minimum(m, d2)

        return _merge_sorted(run_v, run_i, m, lane_iota + t * LANES)

      _, cand = lax.fori_loop(0, num_blocks, blk_body, (inf_v, zero_i))

      pbase = ((cand >> 4) * blk_pts) + (cand & (LANES - 1))
      groups = []
      for i in range(16):
        pidx = pbase + i * LANES
        dx = plsc.load_gather(x_v, [pidx]) - cx
        dy = plsc.load_gather(y_v, [pidx]) - cy
        dz = plsc.load_gather(z_v, [pidx]) - cz
        d2 = dx * dx + dy * dy + dz * dz
        sv, si = plsc.sort_key_val(d2, pidx)
        groups.append((sv, si))
      while len(groups) > 1:
        groups = [_merge_two_sorted(groups[k], groups[k + 1])
                  for k in range(0, len(groups), 2)]
      fin_i = groups[0][1]

      plsc.store_scatter(out_v, [lane_iota, cl_v], fin_i)

    pltpu.sync_copy(
        out_v, out_h.at[pl.ds(b * KNN, KNN), pl.ds(c0, cols_per_worker)])

  return knn


def kernel(xyz, centers):
  b_sz, n_pts, _ = xyz.shape
  k_cen = centers.shape[1]
  knn = _make_knn(b_sz, n_pts, k_cen)
  pts = jnp.transpose(xyz, (2, 0, 1)).reshape(3, b_sz * n_pts)
  cen = jnp.transpose(centers, (2, 0, 1)).reshape(3, b_sz * k_cen)
  out2d = knn(pts[0], pts[1], pts[2], cen[0], cen[1], cen[2])
  return out2d.reshape(b_sz, KNN, k_cen)


# speedup vs baseline: 3.3781x; 1.0875x over previous
import functools

import jax
import jax.numpy as jnp
from jax import lax
from jax.experimental import pallas as pl
from jax.experimental.pallas import tpu as pltpu
from jax.experimental.pallas import tpu_sc as plsc

KNN = 16
LANES = 16
NUM_WORKERS = 32
GRP_PTS = 4096
CHUNK_PTS = 32
N_CHUNK_LANES = 128


def _merge_two_sorted(a, b):
  av, ai = a
  bv, bi = b
  bv = lax.rev(bv, (0,))
  bi = lax.rev(bi, (0,))
  keep = av <= bv
  lo_v = jnp.where(keep, av, bv)
  lo_i = jnp.where(keep, ai, bi)
  out_v, out_i = plsc.sort_key_val(lo_v, lo_i)
  return out_v, out_i


def _tree_select16(groups):
  while len(groups) > 1:
    groups = [_merge_two_sorted(groups[k], groups[k + 1])
              for k in range(0, len(groups), 2)]
  return groups[0]


@functools.lru_cache(maxsize=None)
def _make_tc_chunkmin(b_sz, n_pts, k_cen):
  n_grp = n_pts // GRP_PTS
  assert n_grp * GRP_PTS == n_pts
  k_blk = 64
  assert k_cen % k_blk == 0

  def body(x_ref, y_ref, z_ref, cx_ref, cy_ref, cz_ref, o_ref):
    cxs = [jnp.broadcast_to(cx_ref[0, cv], (8, 128)) for cv in range(8)]
    cys = [jnp.broadcast_to(cy_ref[0, cv], (8, 128)) for cv in range(8)]
    czs = [jnp.broadcast_to(cz_ref[0, cv], (8, 128)) for cv in range(8)]
    m = [None] * 8
    for i in range(GRP_PTS // 128):
      xv = x_ref[0, 0, i, :][None, :]
      yv = y_ref[0, 0, i, :][None, :]
      zv = z_ref[0, 0, i, :][None, :]
      for cv in range(8):
        dx = xv - cxs[cv]
        dy = yv - cys[cv]
        dz = zv - czs[cv]
        d2 = dx * dx + dy * dy + dz * dz
        m[cv] = d2 if m[cv] is None else jnp.minimum(m[cv], d2)
    for cv in range(8):
      o_ref[0, pl.ds(cv * 8, 8), :] = m[cv]

  pt_spec = pl.BlockSpec((1, 1, GRP_PTS // 128, 128),
                         lambda b, j, g: (b, g, 0, 0))
  cen_spec = pl.BlockSpec((1, 8, 8, 1), lambda b, j, g: (b, j, 0, 0))
  return pl.pallas_call(
      body,
      grid=(b_sz, k_cen // k_blk, n_grp),
      in_specs=[pt_spec, pt_spec, pt_spec, cen_spec, cen_spec, cen_spec],
      out_specs=pl.BlockSpec((1, k_blk, N_CHUNK_LANES),
                             lambda b, j, g: (b, j, g)),
      out_shape=jax.ShapeDtypeStruct(
          (b_sz, k_cen, n_grp * N_CHUNK_LANES), jnp.float32),
  )


@functools.lru_cache(maxsize=None)
def _make_sc_select(b_sz, n_pts, k_cen):
  assert NUM_WORKERS % b_sz == 0
  workers_per_batch = NUM_WORKERS // b_sz
  cols_per_worker = k_cen // workers_per_batch
  assert cols_per_worker * workers_per_batch == k_cen
  n_grp = n_pts // GRP_PTS
  n_chunks = n_grp * N_CHUNK_LANES

  mesh = plsc.VectorSubcoreMesh(core_axis_name="c", subcore_axis_name="s")

  @functools.partial(
      pl.kernel,
      out_type=jax.ShapeDtypeStruct((b_sz * KNN, k_cen), jnp.int32),
      mesh=mesh,
      scratch_types=[
          pltpu.VMEM((n_pts,), jnp.float32),
          pltpu.VMEM((n_pts,), jnp.float32),
          pltpu.VMEM((n_pts,), jnp.float32),
          pltpu.VMEM((cols_per_worker,), jnp.float32),
          pltpu.VMEM((cols_per_worker,), jnp.float32),
          pltpu.VMEM((cols_per_worker,), jnp.float32),
          pltpu.VMEM((cols_per_worker * n_chunks,), jnp.float32),
          pltpu.VMEM((KNN, cols_per_worker), jnp.int32),
      ],
      compiler_params=pltpu.CompilerParams(needs_layout_passes=False),
  )
  def knn(x_h, y_h, z_h, cx_h, cy_h, cz_h, cm_h, out_h,
          x_v, y_v, z_v, cx_v, cy_v, cz_v, cm_v, out_v):
    wid = lax.axis_index("s") * 2 + lax.axis_index("c")
    b = wid // workers_per_batch
    c0 = (wid % workers_per_batch) * cols_per_worker

    pltpu.sync_copy(x_h.at[pl.ds(b * n_pts, n_pts)], x_v)
    pltpu.sync_copy(y_h.at[pl.ds(b * n_pts, n_pts)], y_v)
    pltpu.sync_copy(z_h.at[pl.ds(b * n_pts, n_pts)], z_v)
    pltpu.sync_copy(cx_h.at[pl.ds(b * k_cen + c0, cols_per_worker)], cx_v)
    pltpu.sync_copy(cy_h.at[pl.ds(b * k_cen + c0, cols_per_worker)], cy_v)
    pltpu.sync_copy(cz_h.at[pl.ds(b * k_cen + c0, cols_per_worker)], cz_v)
    pltpu.sync_copy(
        cm_h.at[pl.ds((b * k_cen + c0) * n_chunks,
                      cols_per_worker * n_chunks)], cm_v)

    lane_iota = lax.iota(jnp.int32, LANES)

    @pl.loop(0, cols_per_worker)
    def col_loop(cl):
      cl_v = jnp.full((LANES,), cl, jnp.int32)
      cx = plsc.load_gather(cx_v, [cl_v])
      cy = plsc.load_gather(cy_v, [cl_v])
      cz = plsc.load_gather(cz_v, [cl_v])

      cm_base = cl * n_chunks
      groups = []
      for v in range(n_chunks // LANES):
        vals = cm_v[pl.ds(cm_base + v * LANES, LANES)]
        sv, si = plsc.sort_key_val(vals, lane_iota + v * LANES)
        groups.append((sv, si))
      cand = _tree_select16(groups)[1]

      pbase = ((cand >> 7) * GRP_PTS) + (cand & (N_CHUNK_LANES - 1))
      groups = []
      for i in range(CHUNK_PTS):
        pidx = pbase + i * N_CHUNK_LANES
        dx = plsc.load_gather(x_v, [pidx]) - cx
        dy = plsc.load_gather(y_v, [pidx]) - cy
        dz = plsc.load_gather(z_v, [pidx]) - cz
        d2 = dx * dx + dy * dy + dz * dz
        sv, si = plsc.sort_key_val(d2, pidx)
        groups.append((sv, si))
      fin_i = _tree_select16(groups)[1]

      plsc.store_scatter(out_v, [lane_iota, cl_v], fin_i)

    pltpu.sync_copy(
        out_v, out_h.at[pl.ds(b * KNN, KNN), pl.ds(c0, cols_per_worker)])

  return knn


def kernel(xyz, centers):
  b_sz, n_pts, _ = xyz.shape
  k_cen = centers.shape[1]
  pts = jnp.transpose(xyz, (2, 0, 1)).reshape(3, b_sz, n_pts)
  cen = jnp.transpose(centers, (2, 0, 1)).reshape(3, b_sz, k_cen)
  cen4 = cen.reshape(3, b_sz, k_cen // 8, 8, 1)
  pts4 = pts.reshape(3, b_sz, n_pts // GRP_PTS, GRP_PTS // 128, 128)

  tc = _make_tc_chunkmin(b_sz, n_pts, k_cen)
  cm = tc(pts4[0], pts4[1], pts4[2], cen4[0], cen4[1], cen4[2])

  sc = _make_sc_select(b_sz, n_pts, k_cen)
  out2d = sc(pts[0].reshape(-1), pts[1].reshape(-1), pts[2].reshape(-1),
             cen[0].reshape(-1), cen[1].reshape(-1), cen[2].reshape(-1),
             cm.reshape(-1))
  return out2d.reshape(b_sz, KNN, k_cen)


# speedup vs baseline: 5.3243x; 1.5761x over previous
import functools

import jax
import jax.numpy as jnp
from jax import lax
from jax.experimental import pallas as pl
from jax.experimental.pallas import tpu as pltpu
from jax.experimental.pallas import tpu_sc as plsc

KNN = 16
LANES = 16
NUM_WORKERS = 32
GRP_PTS = 4096
CHUNK_PTS = 32
N_CHUNK_LANES = 128


def _merge_two_sorted(a, b):
  av, ai = a
  bv, bi = b
  bv = lax.rev(bv, (0,))
  bi = lax.rev(bi, (0,))
  keep = av <= bv
  lo_v = jnp.where(keep, av, bv)
  lo_i = jnp.where(keep, ai, bi)
  out_v, out_i = plsc.sort_key_val(lo_v, lo_i)
  return out_v, out_i


def _tree_select16(groups):
  while len(groups) > 1:
    groups = [_merge_two_sorted(groups[k], groups[k + 1])
              for k in range(0, len(groups), 2)]
  return groups[0]


@functools.lru_cache(maxsize=None)
def _make_tc_chunkmin(b_sz, n_pts, k_cen):
  n_vregs = n_pts // 1024
  k_blk = 64
  assert k_cen % k_blk == 0

  def body(x_ref, y_ref, z_ref, cx_ref, cy_ref, cz_ref, o_ref):
    @pl.loop(0, k_blk // 8)
    def kblk(kb):
      sx, sy, sz, m = [], [], [], [None] * 8
      for kk in range(8):
        k = kb * 8 + kk
        sx.append(jnp.full((8, 128), cx_ref[0, 0, 0, k], jnp.float32))
        sy.append(jnp.full((8, 128), cy_ref[0, 0, 0, k], jnp.float32))
        sz.append(jnp.full((8, 128), cz_ref[0, 0, 0, k], jnp.float32))
      for i in range(n_vregs):
        xv = x_ref[0, i]
        yv = y_ref[0, i]
        zv = z_ref[0, i]
        for kk in range(8):
          dx = xv - sx[kk]
          dy = yv - sy[kk]
          dz = zv - sz[kk]
          d2 = dx * dx + dy * dy + dz * dz
          m[kk] = d2 if m[kk] is None else jnp.minimum(m[kk], d2)
      for kk in range(8):
        o_ref[0, kb * 8 + kk] = jnp.minimum(m[kk][0:4], m[kk][4:8])

  pt_spec = pl.BlockSpec((1, n_vregs, 8, 128), lambda b, j: (b, 0, 0, 0))
  cen_spec = pl.BlockSpec((1, 1, 1, k_blk), lambda b, j: (b, j, 0, 0),
                          memory_space=pltpu.SMEM)
  return pl.pallas_call(
      body,
      grid=(b_sz, k_cen // k_blk),
      in_specs=[pt_spec, pt_spec, pt_spec, cen_spec, cen_spec, cen_spec],
      out_specs=pl.BlockSpec((1, k_blk, 4, N_CHUNK_LANES),
                             lambda b, j: (b, j, 0, 0)),
      out_shape=jax.ShapeDtypeStruct(
          (b_sz, k_cen, 4, N_CHUNK_LANES), jnp.float32),
  )


@functools.lru_cache(maxsize=None)
def _make_sc_select(b_sz, n_pts, k_cen):
  assert NUM_WORKERS % b_sz == 0
  workers_per_batch = NUM_WORKERS // b_sz
  cols_per_worker = k_cen // workers_per_batch
  assert cols_per_worker * workers_per_batch == k_cen
  n_grp = n_pts // GRP_PTS
  n_chunks = n_grp * N_CHUNK_LANES

  mesh = plsc.VectorSubcoreMesh(core_axis_name="c", subcore_axis_name="s")

  @functools.partial(
      pl.kernel,
      out_type=jax.ShapeDtypeStruct((b_sz * KNN, k_cen), jnp.int32),
      mesh=mesh,
      scratch_types=[
          pltpu.VMEM((n_pts,), jnp.float32),
          pltpu.VMEM((n_pts,), jnp.float32),
          pltpu.VMEM((n_pts,), jnp.float32),
          pltpu.VMEM((cols_per_worker,), jnp.float32),
          pltpu.VMEM((cols_per_worker,), jnp.float32),
          pltpu.VMEM((cols_per_worker,), jnp.float32),
          pltpu.VMEM((cols_per_worker * n_chunks,), jnp.float32),
          pltpu.VMEM((KNN, cols_per_worker), jnp.int32),
      ],
      compiler_params=pltpu.CompilerParams(needs_layout_passes=False),
  )
  def knn(x_h, y_h, z_h, cx_h, cy_h, cz_h, cm_h, out_h,
          x_v, y_v, z_v, cx_v, cy_v, cz_v, cm_v, out_v):
    wid = lax.axis_index("s") * 2 + lax.axis_index("c")
    b = wid // workers_per_batch
    c0 = (wid % workers_per_batch) * cols_per_worker

    pltpu.sync_copy(x_h.at[pl.ds(b * n_pts, n_pts)], x_v)
    pltpu.sync_copy(y_h.at[pl.ds(b * n_pts, n_pts)], y_v)
    pltpu.sync_copy(z_h.at[pl.ds(b * n_pts, n_pts)], z_v)
    pltpu.sync_copy(cx_h.at[pl.ds(b * k_cen + c0, cols_per_worker)], cx_v)
    pltpu.sync_copy(cy_h.at[pl.ds(b * k_cen + c0, cols_per_worker)], cy_v)
    pltpu.sync_copy(cz_h.at[pl.ds(b * k_cen + c0, cols_per_worker)], cz_v)
    pltpu.sync_copy(
        cm_h.at[pl.ds((b * k_cen + c0) * n_chunks,
                      cols_per_worker * n_chunks)], cm_v)

    lane_iota = lax.iota(jnp.int32, LANES)

    @pl.loop(0, cols_per_worker)
    def col_loop(cl):
      cl_v = jnp.full((LANES,), cl, jnp.int32)
      cx = plsc.load_gather(cx_v, [cl_v])
      cy = plsc.load_gather(cy_v, [cl_v])
      cz = plsc.load_gather(cz_v, [cl_v])

      cm_base = cl * n_chunks
      groups = []
      for v in range(n_chunks // LANES):
        vals = cm_v[pl.ds(cm_base + v * LANES, LANES)]
        sv, si = plsc.sort_key_val(vals, lane_iota + v * LANES)
        groups.append((sv, si))
      cand = _tree_select16(groups)[1]

      groups = []
      for i in range(CHUNK_PTS):
        pidx = cand + i * 512
        dx = plsc.load_gather(x_v, [pidx]) - cx
        dy = plsc.load_gather(y_v, [pidx]) - cy
        dz = plsc.load_gather(z_v, [pidx]) - cz
        d2 = dx * dx + dy * dy + dz * dz
        sv, si = plsc.sort_key_val(d2, pidx)
        groups.append((sv, si))
      fin_i = _tree_select16(groups)[1]

      plsc.store_scatter(out_v, [lane_iota, cl_v], fin_i)

    pltpu.sync_copy(
        out_v, out_h.at[pl.ds(b * KNN, KNN), pl.ds(c0, cols_per_worker)])

  return knn


def kernel(xyz, centers):
  b_sz, n_pts, _ = xyz.shape
  k_cen = centers.shape[1]
  pts = jnp.transpose(xyz, (2, 0, 1)).reshape(3, b_sz, n_pts)
  cen = jnp.transpose(centers, (2, 0, 1)).reshape(3, b_sz, k_cen)
  pts4 = pts.reshape(3, b_sz, n_pts // 1024, 8, 128)

  cenb = cen.reshape(3, b_sz, k_cen // 64, 1, 64)
  tc = _make_tc_chunkmin(b_sz, n_pts, k_cen)
  cm = tc(pts4[0], pts4[1], pts4[2], cenb[0], cenb[1], cenb[2])

  sc = _make_sc_select(b_sz, n_pts, k_cen)
  out2d = sc(pts[0].reshape(-1), pts[1].reshape(-1), pts[2].reshape(-1),
             cen[0].reshape(-1), cen[1].reshape(-1), cen[2].reshape(-1),
             cm.reshape(-1))
  return out2d.reshape(b_sz, KNN, k_cen)


# speedup vs baseline: 5.6239x; 1.0563x over previous
import functools

import jax
import jax.numpy as jnp
from jax import lax
from jax.experimental import pallas as pl
from jax.experimental.pallas import tpu as pltpu
from jax.experimental.pallas import tpu_sc as plsc

KNN = 16
LANES = 16
NUM_WORKERS = 32
GRP_PTS = 4096
CHUNK_PTS = 32
N_CHUNK_LANES = 128


def _merge_two_sorted(a, b):
  av, ai = a
  bv, bi = b
  bv = lax.rev(bv, (0,))
  bi = lax.rev(bi, (0,))
  keep = av <= bv
  lo_v = jnp.where(keep, av, bv)
  lo_i = jnp.where(keep, ai, bi)
  out_v, out_i = plsc.sort_key_val(lo_v, lo_i)
  return out_v, out_i


def _tree_select16(groups):
  while len(groups) > 1:
    groups = [_merge_two_sorted(groups[k], groups[k + 1])
              for k in range(0, len(groups), 2)]
  return groups[0]


@functools.lru_cache(maxsize=None)
def _make_tc_chunkmin(b_sz, n_pts, k_cen):
  n_vregs = n_pts // 1024
  k_blk = 64
  assert k_cen % k_blk == 0

  def body(x_ref, y_ref, z_ref, cx_ref, cy_ref, cz_ref, o_ref):
    @pl.loop(0, k_blk // 8)
    def kblk(kb):
      sx, sy, sz, m = [], [], [], [None] * 8
      for kk in range(8):
        k = kb * 8 + kk
        sx.append(jnp.full((8, 128), cx_ref[0, 0, 0, k], jnp.float32))
        sy.append(jnp.full((8, 128), cy_ref[0, 0, 0, k], jnp.float32))
        sz.append(jnp.full((8, 128), cz_ref[0, 0, 0, k], jnp.float32))
      for i in range(n_vregs):
        xv = x_ref[0, i]
        yv = y_ref[0, i]
        zv = z_ref[0, i]
        for kk in range(8):
          dx = xv - sx[kk]
          dy = yv - sy[kk]
          dz = zv - sz[kk]
          d2 = dx * dx + dy * dy + dz * dz
          m[kk] = d2 if m[kk] is None else jnp.minimum(m[kk], d2)
      for kk in range(8):
        o_ref[0, kb * 8 + kk] = jnp.minimum(m[kk][0:4], m[kk][4:8])

  pt_spec = pl.BlockSpec((1, n_vregs, 8, 128), lambda b, j: (b, 0, 0, 0))
  cen_spec = pl.BlockSpec((1, 1, 1, k_blk), lambda b, j: (b, j, 0, 0),
                          memory_space=pltpu.SMEM)
  return pl.pallas_call(
      body,
      grid=(b_sz, k_cen // k_blk),
      in_specs=[pt_spec, pt_spec, pt_spec, cen_spec, cen_spec, cen_spec],
      out_specs=pl.BlockSpec((1, k_blk, 4, N_CHUNK_LANES),
                             lambda b, j: (b, j, 0, 0)),
      out_shape=jax.ShapeDtypeStruct(
          (b_sz, k_cen, 4, N_CHUNK_LANES), jnp.float32),
  )


@functools.lru_cache(maxsize=None)
def _make_sc_select(b_sz, n_pts, k_cen):
  assert NUM_WORKERS % b_sz == 0
  workers_per_batch = NUM_WORKERS // b_sz
  cols_per_worker = k_cen // workers_per_batch
  assert cols_per_worker * workers_per_batch == k_cen
  n_grp = n_pts // GRP_PTS
  n_chunks = n_grp * N_CHUNK_LANES

  mesh = plsc.VectorSubcoreMesh(core_axis_name="c", subcore_axis_name="s")

  @functools.partial(
      pl.kernel,
      out_type=jax.ShapeDtypeStruct((b_sz * k_cen * KNN,), jnp.int32),
      mesh=mesh,
      scratch_types=[
          pltpu.VMEM((n_pts,), jnp.float32),
          pltpu.VMEM((n_pts,), jnp.float32),
          pltpu.VMEM((n_pts,), jnp.float32),
          pltpu.VMEM((cols_per_worker,), jnp.float32),
          pltpu.VMEM((cols_per_worker,), jnp.float32),
          pltpu.VMEM((cols_per_worker,), jnp.float32),
          pltpu.VMEM((cols_per_worker * n_chunks,), jnp.float32),
          pltpu.VMEM((cols_per_worker * KNN,), jnp.int32),
      ],
      compiler_params=pltpu.CompilerParams(needs_layout_passes=False),
  )
  def knn(x_h, y_h, z_h, cx_h, cy_h, cz_h, cm_h, out_h,
          x_v, y_v, z_v, cx_v, cy_v, cz_v, cm_v, out_v):
    wid = lax.axis_index("s") * 2 + lax.axis_index("c")
    b = wid // workers_per_batch
    c0 = (wid % workers_per_batch) * cols_per_worker

    pltpu.sync_copy(x_h.at[pl.ds(b * n_pts, n_pts)], x_v)
    pltpu.sync_copy(y_h.at[pl.ds(b * n_pts, n_pts)], y_v)
    pltpu.sync_copy(z_h.at[pl.ds(b * n_pts, n_pts)], z_v)
    pltpu.sync_copy(cx_h.at[pl.ds(b * k_cen + c0, cols_per_worker)], cx_v)
    pltpu.sync_copy(cy_h.at[pl.ds(b * k_cen + c0, cols_per_worker)], cy_v)
    pltpu.sync_copy(cz_h.at[pl.ds(b * k_cen + c0, cols_per_worker)], cz_v)
    pltpu.sync_copy(
        cm_h.at[pl.ds((b * k_cen + c0) * n_chunks,
                      cols_per_worker * n_chunks)], cm_v)

    lane_iota = lax.iota(jnp.int32, LANES)

    @pl.loop(0, cols_per_worker)
    def col_loop(cl):
      cl_v = jnp.full((LANES,), cl, jnp.int32)
      cx = plsc.load_gather(cx_v, [cl_v])
      cy = plsc.load_gather(cy_v, [cl_v])
      cz = plsc.load_gather(cz_v, [cl_v])

      cm_base = cl * n_chunks
      groups = []
      for v in range(n_chunks // LANES):
        vals = cm_v[pl.ds(cm_base + v * LANES, LANES)]
        sv, si = plsc.sort_key_val(vals, lane_iota + v * LANES)
        groups.append((sv, si))
      cand = _tree_select16(groups)[1]

      groups = []
      for i in range(CHUNK_PTS):
        pidx = cand + i * 512
        dx = plsc.load_gather(x_v, [pidx]) - cx
        dy = plsc.load_gather(y_v, [pidx]) - cy
        dz = plsc.load_gather(z_v, [pidx]) - cz
        d2 = dx * dx + dy * dy + dz * dz
        sv, si = plsc.sort_key_val(d2, pidx)
        groups.append((sv, si))
      fin_i = _tree_select16(groups)[1]

      out_v[pl.ds(cl * KNN, KNN)] = fin_i

    pltpu.sync_copy(
        out_v,
        out_h.at[pl.ds((b * k_cen + c0) * KNN, cols_per_worker * KNN)])

  return knn


def _run_half(pts, cen, n_pts, k_cen):
  b_sz = pts.shape[1]
  pts4 = pts.reshape(3, b_sz, n_pts // 1024, 8, 128)
  cenb = cen.reshape(3, b_sz, k_cen // 64, 1, 64)
  tc = _make_tc_chunkmin(b_sz, n_pts, k_cen)
  cm = tc(pts4[0], pts4[1], pts4[2], cenb[0], cenb[1], cenb[2])
  sc = _make_sc_select(b_sz, n_pts, k_cen)
  out = sc(pts[0].reshape(-1), pts[1].reshape(-1), pts[2].reshape(-1),
           cen[0].reshape(-1), cen[1].reshape(-1), cen[2].reshape(-1),
           cm.reshape(-1))
  return jnp.swapaxes(out.reshape(b_sz, k_cen, KNN), 1, 2)


def kernel(xyz, centers):
  b_sz, n_pts, _ = xyz.shape
  k_cen = centers.shape[1]
  pts = jnp.transpose(xyz, (2, 0, 1)).reshape(3, b_sz, n_pts)
  cen = jnp.transpose(centers, (2, 0, 1)).reshape(3, b_sz, k_cen)
  if b_sz % 2:
    return _run_half(pts, cen, n_pts, k_cen)
  h = b_sz // 2
  out0 = _run_half(pts[:, :h], cen[:, :h], n_pts, k_cen)
  out1 = _run_half(pts[:, h:], cen[:, h:], n_pts, k_cen)
  return jnp.concatenate([out0, out1], axis=0)
